# baseline (speedup 1.0000x reference)
import numpy as np

import jax
import jax.numpy as jnp
from jax.experimental import pallas as pl

B, T = 1024, 1024
CONV_DIM, MOE_DIM, FF_DIM, E, NUM_CLASSES = 64, 512, 1024, 6, 10

_BF = jnp.bfloat16

_R = 64

_E1 = np.zeros((24, 8, 5), dtype=np.float32)
for _j in range(8):
    for _k in range(5):
        _E1[2 * _j + _k, _j, _k] = 1.0

_EA = np.zeros((8, 4, 5), dtype=np.float32)
for _q in range(8):
    for _r in range(4):
        _k = _q + 2 - 2 * _r
        if 0 <= _k < 5:
            _EA[_q, _r, _k] = 1.0
_EB = np.zeros((2, 4, 5), dtype=np.float32)
for _q in range(2):
    for _r in range(4):
        _k = _q - 2 * _r
        if 0 <= _k < 5:
            _EB[_q, _r, _k] = 1.0
_EC = np.zeros((1, 4, 5), dtype=np.float32)
for _r in range(4):
    _k = 10 - 2 * _r
    if 0 <= _k < 5:
        _EC[0, _r, _k] = 1.0


def _fused_kernel(xa_ref, c1_ref, b1big_ref, w2f_ref, b2big_ref,
                  projwt_ref, projb_ref, routwt_ref, routb_ref,
                  w1_ref, b1_ref, w2_ref, b2_ref, clswt_ref, clsb_ref,
                  y_ref):
    r = _R
    xa = xa_ref[...]
    xwb = jnp.concatenate([xa[:, 0:64, :], xa[:, 1:65, 0:8]],
                          axis=2).reshape(r * 64, 24)
    h1 = jnp.dot(xwb, c1_ref[...],
                 preferred_element_type=jnp.float32) + b1big_ref[...]
    a = jnp.maximum(h1, 0.0).astype(_BF).reshape(r, 64, 8 * CONV_DIM)
    zb = jnp.zeros((r, 1, 8 * CONV_DIM), dtype=_BF)
    prev = jnp.concatenate([zb, a[:, 0:63, :]], axis=1)[:, :, 6 * CONV_DIM:]
    nxt = jnp.concatenate([a[:, 1:64, :], zb], axis=1)[:, :, 0:CONV_DIM]
    lhs = jnp.concatenate([a, prev, nxt], axis=2)
    h2 = jnp.dot(lhs.reshape(r * 64, 11 * CONV_DIM), w2f_ref[...],
                 preferred_element_type=jnp.float32) + b2big_ref[...]
    h2 = jnp.maximum(h2, 0.0)
    h2s = jnp.sum(h2.reshape(r, 64, 4 * CONV_DIM), axis=1)
    m = (h2s[:, 0:64] + h2s[:, 64:128] + h2s[:, 128:192]
         + h2s[:, 192:256]) * (1.0 / 256.0)
    h = jnp.dot(m.astype(_BF), projwt_ref[...],
                preferred_element_type=jnp.float32) + projb_ref[...]
    logits = jnp.dot(h.astype(_BF), routwt_ref[...],
                     preferred_element_type=jnp.float32) + routb_ref[...]
    mx = jnp.max(logits, axis=-1, keepdims=True)
    e = jnp.exp(logits - mx)
    p = e / jnp.sum(e, axis=-1, keepdims=True)
    iota = jax.lax.broadcasted_iota(jnp.int32, (r, 8), 1)
    i1 = jnp.argmax(p, axis=-1)[:, None]
    m1 = iota == i1
    p_masked = jnp.where(m1, -1.0, p)
    i2 = jnp.argmax(p_masked, axis=-1)[:, None]
    m2 = iota == i2
    g = jnp.where(m1 | m2, p, 0.0)
    hb = h.astype(_BF)
    acc = jnp.zeros((r, MOE_DIM), dtype=jnp.float32)
    for j in range(E):
        hid = jnp.dot(hb, w1_ref[j],
                      preferred_element_type=jnp.float32) + b1_ref[j][None, :]
        hid = jnp.maximum(hid, 0.0)
        eo = jnp.dot(hid.astype(_BF), w2_ref[j],
                     preferred_element_type=jnp.float32) + b2_ref[j][None, :]
        acc = acc + g[:, j:j + 1] * eo
    y_ref[...] = jnp.dot(acc.astype(_BF), clswt_ref[...],
                         preferred_element_type=jnp.float32) + clsb_ref[...]


def kernel(x, conv1_w, conv1_b, conv2_w, conv2_b, proj_w, proj_b,
           router_w, router_b, w1, b1, w2, b2, cls_w, cls_b):
    f32 = jnp.float32
    x = x.astype(f32)
    xpad = jnp.pad(x, ((0, 0), (2, 14))).astype(_BF)
    xa = xpad.reshape(B, 65, 16)
    c1 = jnp.einsum('pjk,ck->pjc', jnp.asarray(_E1),
                    conv1_w[:, 0, :]).reshape(24, 8 * CONV_DIM).astype(_BF)
    b1big = jnp.tile(conv1_b, 8)[None, :]
    w2f = jnp.concatenate([
        jnp.einsum('qrk,dck->qcrd', jnp.asarray(_EA),
                   conv2_w).reshape(8 * CONV_DIM, 4 * CONV_DIM),
        jnp.einsum('qrk,dck->qcrd', jnp.asarray(_EB),
                   conv2_w).reshape(2 * CONV_DIM, 4 * CONV_DIM),
        jnp.einsum('qrk,dck->qcrd', jnp.asarray(_EC),
                   conv2_w).reshape(CONV_DIM, 4 * CONV_DIM),
    ], axis=0).astype(_BF)
    b2big = jnp.tile(conv2_b, 4)[None, :]
    projwt = proj_w.T.astype(_BF)
    projbr = proj_b[None, :]
    routwt = jnp.pad(router_w.T, ((0, 0), (0, 2))).astype(_BF)
    routb = jnp.pad(router_b, (0, 2), constant_values=-1e30)[None, :]

    y = pl.pallas_call(
        _fused_kernel,
        grid=(B // _R,),
        in_specs=[
            pl.BlockSpec((_R, 65, 16), lambda i: (i, 0, 0)),
            pl.BlockSpec((24, 8 * CONV_DIM), lambda i: (0, 0)),
            pl.BlockSpec((1, 8 * CONV_DIM), lambda i: (0, 0)),
            pl.BlockSpec((11 * CONV_DIM, 4 * CONV_DIM), lambda i: (0, 0)),
            pl.BlockSpec((1, 4 * CONV_DIM), lambda i: (0, 0)),
            pl.BlockSpec((CONV_DIM, MOE_DIM), lambda i: (0, 0)),
            pl.BlockSpec((1, MOE_DIM), lambda i: (0, 0)),
            pl.BlockSpec((MOE_DIM, 8), lambda i: (0, 0)),
            pl.BlockSpec((1, 8), lambda i: (0, 0)),
            pl.BlockSpec((E, MOE_DIM, FF_DIM), lambda i: (0, 0, 0)),
            pl.BlockSpec((E, FF_DIM), lambda i: (0, 0)),
            pl.BlockSpec((E, FF_DIM, MOE_DIM), lambda i: (0, 0, 0)),
            pl.BlockSpec((E, MOE_DIM), lambda i: (0, 0)),
            pl.BlockSpec((MOE_DIM, NUM_CLASSES), lambda i: (0, 0)),
            pl.BlockSpec((1, NUM_CLASSES), lambda i: (0, 0)),
        ],
        out_specs=pl.BlockSpec((_R, NUM_CLASSES), lambda i: (i, 0)),
        out_shape=jax.ShapeDtypeStruct((B, NUM_CLASSES), f32),
    )(xa, c1, b1big, w2f, b2big, projwt, projbr, routwt, routb,
      w1.astype(_BF), b1, w2.astype(_BF), b2,
      cls_w.T.astype(_BF), cls_b[None, :])
    return y


# speedup vs baseline: 1.4803x; 1.4803x over previous
import numpy as np

import jax
import jax.numpy as jnp
from jax.experimental import pallas as pl

B, T = 1024, 1024
CONV_DIM, MOE_DIM, FF_DIM, E, NUM_CLASSES = 64, 512, 1024, 6, 10

_BF = jnp.bfloat16

_R1 = 64
_R2 = 512

_E1 = np.zeros((24, 8, 5), dtype=np.float32)
for _j in range(8):
    for _k in range(5):
        _E1[2 * _j + _k, _j, _k] = 1.0

_EA = np.zeros((8, 4, 5), dtype=np.float32)
for _q in range(8):
    for _r in range(4):
        _k = _q + 2 - 2 * _r
        if 0 <= _k < 5:
            _EA[_q, _r, _k] = 1.0
_EB = np.zeros((2, 4, 5), dtype=np.float32)
for _q in range(2):
    for _r in range(4):
        _k = _q - 2 * _r
        if 0 <= _k < 5:
            _EB[_q, _r, _k] = 1.0
_EC = np.zeros((1, 4, 5), dtype=np.float32)
for _r in range(4):
    _k = 10 - 2 * _r
    if 0 <= _k < 5:
        _EC[0, _r, _k] = 1.0


def _frontend_kernel(xa_ref, c1_ref, b1big_ref, w2f_ref, b2big_ref,
                     projwt_ref, projb_ref, routwt_ref, routb_ref,
                     h_ref, g_ref):
    r = _R1
    xa = xa_ref[...]
    xwb = jnp.concatenate([xa[:, 0:64, :], xa[:, 1:65, 0:8]],
                          axis=2).reshape(r * 64, 24)
    h1 = jnp.dot(xwb, c1_ref[...],
                 preferred_element_type=jnp.float32) + b1big_ref[...]
    a = jnp.maximum(h1, 0.0).astype(_BF).reshape(r, 64, 8 * CONV_DIM)
    zb = jnp.zeros((r, 1, 8 * CONV_DIM), dtype=_BF)
    prev = jnp.concatenate([zb, a[:, 0:63, :]], axis=1)[:, :, 6 * CONV_DIM:]
    nxt = jnp.concatenate([a[:, 1:64, :], zb], axis=1)[:, :, 0:CONV_DIM]
    lhs = jnp.concatenate([a, prev, nxt], axis=2)
    h2 = jnp.dot(lhs.reshape(r * 64, 11 * CONV_DIM), w2f_ref[...],
                 preferred_element_type=jnp.float32) + b2big_ref[...]
    h2 = jnp.maximum(h2, 0.0)
    h2s = jnp.sum(h2.reshape(r, 64, 4 * CONV_DIM), axis=1)
    m = (h2s[:, 0:64] + h2s[:, 64:128] + h2s[:, 128:192]
         + h2s[:, 192:256]) * (1.0 / 256.0)
    h = jnp.dot(m.astype(_BF), projwt_ref[...],
                preferred_element_type=jnp.float32) + projb_ref[...]
    logits = jnp.dot(h.astype(_BF), routwt_ref[...],
                     preferred_element_type=jnp.float32) + routb_ref[...]
    mx = jnp.max(logits, axis=-1, keepdims=True)
    e = jnp.exp(logits - mx)
    p = e / jnp.sum(e, axis=-1, keepdims=True)
    iota = jax.lax.broadcasted_iota(jnp.int32, (r, 8), 1)
    i1 = jnp.argmax(p, axis=-1)[:, None]
    m1 = iota == i1
    p_masked = jnp.where(m1, -1.0, p)
    i2 = jnp.argmax(p_masked, axis=-1)[:, None]
    m2 = iota == i2
    g = jnp.where(m1 | m2, p, 0.0)
    h_ref[...] = h
    g_ref[...] = g


def _moe_kernel(h_ref, g_ref, w1_ref, b1_ref, w2_ref, b2_ref,
                clswt_ref, clsb_ref, y_ref):
    h = h_ref[...].astype(_BF)
    g = g_ref[...]
    acc = jnp.zeros((_R2, MOE_DIM), dtype=jnp.float32)
    for j in range(E):
        hid = jnp.dot(h, w1_ref[j],
                      preferred_element_type=jnp.float32) + b1_ref[j][None, :]
        hid = jnp.maximum(hid, 0.0)
        eo = jnp.dot(hid.astype(_BF), w2_ref[j],
                     preferred_element_type=jnp.float32) + b2_ref[j][None, :]
        acc = acc + g[:, j:j + 1] * eo
    y_ref[...] = jnp.dot(acc.astype(_BF), clswt_ref[...],
                         preferred_element_type=jnp.float32) + clsb_ref[...]


def kernel(x, conv1_w, conv1_b, conv2_w, conv2_b, proj_w, proj_b,
           router_w, router_b, w1, b1, w2, b2, cls_w, cls_b):
    f32 = jnp.float32
    x = x.astype(f32)
    xpad = jnp.pad(x, ((0, 0), (2, 14))).astype(_BF)
    xa = xpad.reshape(B, 65, 16)
    c1 = jnp.einsum('pjk,ck->pjc', jnp.asarray(_E1),
                    conv1_w[:, 0, :]).reshape(24, 8 * CONV_DIM).astype(_BF)
    b1big = jnp.tile(conv1_b, 8)[None, :]
    w2f = jnp.concatenate([
        jnp.einsum('qrk,dck->qcrd', jnp.asarray(_EA),
                   conv2_w).reshape(8 * CONV_DIM, 4 * CONV_DIM),
        jnp.einsum('qrk,dck->qcrd', jnp.asarray(_EB),
                   conv2_w).reshape(2 * CONV_DIM, 4 * CONV_DIM),
        jnp.einsum('qrk,dck->qcrd', jnp.asarray(_EC),
                   conv2_w).reshape(CONV_DIM, 4 * CONV_DIM),
    ], axis=0).astype(_BF)
    b2big = jnp.tile(conv2_b, 4)[None, :]
    projwt = proj_w.T.astype(_BF)
    projbr = proj_b[None, :]
    routwt = jnp.pad(router_w.T, ((0, 0), (0, 2))).astype(_BF)
    routb = jnp.pad(router_b, (0, 2), constant_values=-1e30)[None, :]

    grid1 = B // _R1
    h, g = pl.pallas_call(
        _frontend_kernel,
        grid=(grid1,),
        in_specs=[
            pl.BlockSpec((_R1, 65, 16), lambda i: (i, 0, 0)),
            pl.BlockSpec((24, 8 * CONV_DIM), lambda i: (0, 0)),
            pl.BlockSpec((1, 8 * CONV_DIM), lambda i: (0, 0)),
            pl.BlockSpec((11 * CONV_DIM, 4 * CONV_DIM), lambda i: (0, 0)),
            pl.BlockSpec((1, 4 * CONV_DIM), lambda i: (0, 0)),
            pl.BlockSpec((CONV_DIM, MOE_DIM), lambda i: (0, 0)),
            pl.BlockSpec((1, MOE_DIM), lambda i: (0, 0)),
            pl.BlockSpec((MOE_DIM, 8), lambda i: (0, 0)),
            pl.BlockSpec((1, 8), lambda i: (0, 0)),
        ],
        out_specs=[
            pl.BlockSpec((_R1, MOE_DIM), lambda i: (i, 0)),
            pl.BlockSpec((_R1, 8), lambda i: (i, 0)),
        ],
        out_shape=[
            jax.ShapeDtypeStruct((B, MOE_DIM), f32),
            jax.ShapeDtypeStruct((B, 8), f32),
        ],
    )(xa, c1, b1big, w2f, b2big, projwt, projbr, routwt, routb)

    grid2 = B // _R2
    y = pl.pallas_call(
        _moe_kernel,
        grid=(grid2,),
        in_specs=[
            pl.BlockSpec((_R2, MOE_DIM), lambda i: (i, 0)),
            pl.BlockSpec((_R2, 8), lambda i: (i, 0)),
            pl.BlockSpec((E, MOE_DIM, FF_DIM), lambda i: (0, 0, 0)),
            pl.BlockSpec((E, FF_DIM), lambda i: (0, 0)),
            pl.BlockSpec((E, FF_DIM, MOE_DIM), lambda i: (0, 0, 0)),
            pl.BlockSpec((E, MOE_DIM), lambda i: (0, 0)),
            pl.BlockSpec((MOE_DIM, NUM_CLASSES), lambda i: (0, 0)),
            pl.BlockSpec((1, NUM_CLASSES), lambda i: (0, 0)),
        ],
        out_specs=pl.BlockSpec((_R2, NUM_CLASSES), lambda i: (i, 0)),
        out_shape=jax.ShapeDtypeStruct((B, NUM_CLASSES), f32),
    )(h, g, w1.astype(_BF), b1, w2.astype(_BF), b2,
      cls_w.T.astype(_BF), cls_b[None, :])
    return y


# speedup vs baseline: 1.6427x; 1.1097x over previous
import numpy as np

import jax
import jax.numpy as jnp
from jax.experimental import pallas as pl

B, T = 1024, 1024
CONV_DIM, MOE_DIM, FF_DIM, E, NUM_CLASSES = 64, 512, 1024, 6, 10

_BF = jnp.bfloat16

_R1 = 64
_R2 = 512

_E1 = np.zeros((24, 8, 5), dtype=np.float32)
for _j in range(8):
    for _k in range(5):
        _E1[2 * _j + _k, _j, _k] = 1.0

_EA = np.zeros((8, 4, 5), dtype=np.float32)
for _q in range(8):
    for _r in range(4):
        _k = _q + 2 - 2 * _r
        if 0 <= _k < 5:
            _EA[_q, _r, _k] = 1.0
_EB = np.zeros((2, 4, 5), dtype=np.float32)
for _q in range(2):
    for _r in range(4):
        _k = _q - 2 * _r
        if 0 <= _k < 5:
            _EB[_q, _r, _k] = 1.0
_EC = np.zeros((1, 4, 5), dtype=np.float32)
for _r in range(4):
    _k = 10 - 2 * _r
    if 0 <= _k < 5:
        _EC[0, _r, _k] = 1.0


def _frontend_kernel(xa_ref, c1_ref, b1big_ref, w2f_ref, b2big_ref,
                     projwt_ref, projb_ref, routwt_ref, routb_ref,
                     h_ref, g_ref):
    r = _R1
    xa = xa_ref[...]
    xwb = jnp.concatenate([xa[:, 0:64, :], xa[:, 1:65, 0:8]],
                          axis=2).reshape(r * 64, 24)
    h1 = jnp.dot(xwb, c1_ref[...],
                 preferred_element_type=jnp.float32) + b1big_ref[...]
    a = jnp.maximum(h1, 0.0).astype(_BF).reshape(r, 64, 8 * CONV_DIM)
    zb = jnp.zeros((r, 1, 8 * CONV_DIM), dtype=_BF)
    prev = jnp.concatenate([zb, a[:, 0:63, :]], axis=1)[:, :, 6 * CONV_DIM:]
    nxt = jnp.concatenate([a[:, 1:64, :], zb], axis=1)[:, :, 0:CONV_DIM]
    lhs = jnp.concatenate([a, prev, nxt], axis=2)
    h2 = jnp.dot(lhs.reshape(r * 64, 11 * CONV_DIM), w2f_ref[...],
                 preferred_element_type=jnp.float32) + b2big_ref[...]
    h2 = jnp.maximum(h2, 0.0)
    h2s = jnp.sum(h2.reshape(r, 64, 4 * CONV_DIM), axis=1)
    m = (h2s[:, 0:64] + h2s[:, 64:128] + h2s[:, 128:192]
         + h2s[:, 192:256]) * (1.0 / 256.0)
    h = jnp.dot(m.astype(_BF), projwt_ref[...],
                preferred_element_type=jnp.float32) + projb_ref[...]
    logits = jnp.dot(h.astype(_BF), routwt_ref[...],
                     preferred_element_type=jnp.float32) + routb_ref[...]
    mx = jnp.max(logits, axis=-1, keepdims=True)
    e = jnp.exp(logits - mx)
    p = e / jnp.sum(e, axis=-1, keepdims=True)
    iota = jax.lax.broadcasted_iota(jnp.int32, (r, 8), 1)
    i1 = jnp.argmax(p, axis=-1)[:, None]
    m1 = iota == i1
    p_masked = jnp.where(m1, -1.0, p)
    i2 = jnp.argmax(p_masked, axis=-1)[:, None]
    m2 = iota == i2
    g = jnp.where(m1 | m2, p, 0.0)
    h_ref[...] = h
    g_ref[...] = g


def _moe_kernel(h_ref, g_ref, w1_ref, b1_ref, w2_ref, b2_ref,
                clswt_ref, clsb_ref, y_ref):
    h = h_ref[...].astype(_BF)
    g = g_ref[...]
    acc = jnp.zeros((_R2, MOE_DIM), dtype=jnp.float32)
    for j in range(E):
        hid = jnp.dot(h, w1_ref[j].astype(_BF),
                      preferred_element_type=jnp.float32) + b1_ref[j][None, :]
        hid = jnp.maximum(hid, 0.0)
        eo = jnp.dot(hid.astype(_BF), w2_ref[j].astype(_BF),
                     preferred_element_type=jnp.float32) + b2_ref[j][None, :]
        acc = acc + g[:, j:j + 1] * eo
    y_ref[...] = jnp.dot(acc.astype(_BF), clswt_ref[...],
                         preferred_element_type=jnp.float32) + clsb_ref[...]


def kernel(x, conv1_w, conv1_b, conv2_w, conv2_b, proj_w, proj_b,
           router_w, router_b, w1, b1, w2, b2, cls_w, cls_b):
    f32 = jnp.float32
    x = x.astype(f32)
    xpad = jnp.pad(x, ((0, 0), (2, 14))).astype(_BF)
    xa = xpad.reshape(B, 65, 16)
    c1 = jnp.einsum('pjk,ck->pjc', jnp.asarray(_E1),
                    conv1_w[:, 0, :]).reshape(24, 8 * CONV_DIM).astype(_BF)
    b1big = jnp.tile(conv1_b, 8)[None, :]
    w2f = jnp.concatenate([
        jnp.einsum('qrk,dck->qcrd', jnp.asarray(_EA),
                   conv2_w).reshape(8 * CONV_DIM, 4 * CONV_DIM),
        jnp.einsum('qrk,dck->qcrd', jnp.asarray(_EB),
                   conv2_w).reshape(2 * CONV_DIM, 4 * CONV_DIM),
        jnp.einsum('qrk,dck->qcrd', jnp.asarray(_EC),
                   conv2_w).reshape(CONV_DIM, 4 * CONV_DIM),
    ], axis=0).astype(_BF)
    b2big = jnp.tile(conv2_b, 4)[None, :]
    projwt = proj_w.T.astype(_BF)
    projbr = proj_b[None, :]
    routwt = jnp.pad(router_w.T, ((0, 0), (0, 2))).astype(_BF)
    routb = jnp.pad(router_b, (0, 2), constant_values=-1e30)[None, :]

    grid1 = B // _R1
    h, g = pl.pallas_call(
        _frontend_kernel,
        grid=(grid1,),
        in_specs=[
            pl.BlockSpec((_R1, 65, 16), lambda i: (i, 0, 0)),
            pl.BlockSpec((24, 8 * CONV_DIM), lambda i: (0, 0)),
            pl.BlockSpec((1, 8 * CONV_DIM), lambda i: (0, 0)),
            pl.BlockSpec((11 * CONV_DIM, 4 * CONV_DIM), lambda i: (0, 0)),
            pl.BlockSpec((1, 4 * CONV_DIM), lambda i: (0, 0)),
            pl.BlockSpec((CONV_DIM, MOE_DIM), lambda i: (0, 0)),
            pl.BlockSpec((1, MOE_DIM), lambda i: (0, 0)),
            pl.BlockSpec((MOE_DIM, 8), lambda i: (0, 0)),
            pl.BlockSpec((1, 8), lambda i: (0, 0)),
        ],
        out_specs=[
            pl.BlockSpec((_R1, MOE_DIM), lambda i: (i, 0)),
            pl.BlockSpec((_R1, 8), lambda i: (i, 0)),
        ],
        out_shape=[
            jax.ShapeDtypeStruct((B, MOE_DIM), f32),
            jax.ShapeDtypeStruct((B, 8), f32),
        ],
    )(xa, c1, b1big, w2f, b2big, projwt, projbr, routwt, routb)

    grid2 = B // _R2
    y = pl.pallas_call(
        _moe_kernel,
        grid=(grid2,),
        in_specs=[
            pl.BlockSpec((_R2, MOE_DIM), lambda i: (i, 0)),
            pl.BlockSpec((_R2, 8), lambda i: (i, 0)),
            pl.BlockSpec((E, MOE_DIM, FF_DIM), lambda i: (0, 0, 0)),
            pl.BlockSpec((E, FF_DIM), lambda i: (0, 0)),
            pl.BlockSpec((E, FF_DIM, MOE_DIM), lambda i: (0, 0, 0)),
            pl.BlockSpec((E, MOE_DIM), lambda i: (0, 0)),
            pl.BlockSpec((MOE_DIM, NUM_CLASSES), lambda i: (0, 0)),
            pl.BlockSpec((1, NUM_CLASSES), lambda i: (0, 0)),
        ],
        out_specs=pl.BlockSpec((_R2, NUM_CLASSES), lambda i: (i, 0)),
        out_shape=jax.ShapeDtypeStruct((B, NUM_CLASSES), f32),
    )(h, g, w1, b1, w2, b2,
      cls_w.T.astype(_BF), cls_b[None, :])
    return y


# speedup vs baseline: 1.7467x; 1.0633x over previous
import numpy as np

import jax
import jax.numpy as jnp
from jax.experimental import pallas as pl

B, T = 1024, 1024
CONV_DIM, MOE_DIM, FF_DIM, E, NUM_CLASSES = 64, 512, 1024, 6, 10

_BF = jnp.bfloat16

_R1 = 128
_R2 = 512

_E1 = np.zeros((24, 8, 5), dtype=np.float32)
for _j in range(8):
    for _k in range(5):
        _E1[2 * _j + _k, _j, _k] = 1.0

_EA = np.zeros((8, 4, 5), dtype=np.float32)
for _q in range(8):
    for _r in range(4):
        _k = _q + 2 - 2 * _r
        if 0 <= _k < 5:
            _EA[_q, _r, _k] = 1.0
_EB = np.zeros((2, 4, 5), dtype=np.float32)
for _q in range(2):
    for _r in range(4):
        _k = _q - 2 * _r
        if 0 <= _k < 5:
            _EB[_q, _r, _k] = 1.0
_EC = np.zeros((1, 4, 5), dtype=np.float32)
for _r in range(4):
    _k = 10 - 2 * _r
    if 0 <= _k < 5:
        _EC[0, _r, _k] = 1.0


def _frontend_kernel(xa_ref, c1_ref, b1big_ref, w2f_ref, b2big_ref,
                     projwt_ref, projb_ref, routwt_ref, routb_ref,
                     h_ref, g_ref):
    r = _R1
    xa = xa_ref[...]
    xwb = jnp.concatenate([xa[:, 0:64, :], xa[:, 1:65, 0:8]],
                          axis=2).reshape(r * 64, 24)
    h1 = jnp.dot(xwb, c1_ref[...],
                 preferred_element_type=jnp.float32) + b1big_ref[...]
    a = jnp.maximum(h1, 0.0).astype(_BF).reshape(r, 64, 8 * CONV_DIM)
    zb = jnp.zeros((r, 1, 8 * CONV_DIM), dtype=_BF)
    prev = jnp.concatenate([zb, a[:, 0:63, :]], axis=1)[:, :, 6 * CONV_DIM:]
    nxt = jnp.concatenate([a[:, 1:64, :], zb], axis=1)[:, :, 0:CONV_DIM]
    lhs = jnp.concatenate([a, prev, nxt], axis=2)
    h2 = jnp.dot(lhs.reshape(r * 64, 11 * CONV_DIM), w2f_ref[...],
                 preferred_element_type=jnp.float32) + b2big_ref[...]
    h2 = jnp.maximum(h2, 0.0)
    h2s = jnp.sum(h2.reshape(r, 64, 4 * CONV_DIM), axis=1)
    m = (h2s[:, 0:64] + h2s[:, 64:128] + h2s[:, 128:192]
         + h2s[:, 192:256]) * (1.0 / 256.0)
    h = jnp.dot(m.astype(_BF), projwt_ref[...],
                preferred_element_type=jnp.float32) + projb_ref[...]
    logits = jnp.dot(h.astype(_BF), routwt_ref[...],
                     preferred_element_type=jnp.float32) + routb_ref[...]
    mx = jnp.max(logits, axis=-1, keepdims=True)
    e = jnp.exp(logits - mx)
    p = e / jnp.sum(e, axis=-1, keepdims=True)
    iota = jax.lax.broadcasted_iota(jnp.int32, (r, 8), 1)
    i1 = jnp.argmax(p, axis=-1)[:, None]
    m1 = iota == i1
    p_masked = jnp.where(m1, -1.0, p)
    i2 = jnp.argmax(p_masked, axis=-1)[:, None]
    m2 = iota == i2
    g = jnp.where(m1 | m2, p, 0.0)
    h_ref[...] = h
    g_ref[...] = g


def _moe_kernel(h_ref, g_ref, w1_ref, b1_ref, w2_ref, b2_ref,
                clswt_ref, clsb_ref, y_ref):
    h = h_ref[...].astype(_BF)
    g = g_ref[...]
    acc = jnp.zeros((_R2, MOE_DIM), dtype=jnp.float32)
    for j in range(E):
        hid = jnp.dot(h, w1_ref[j].astype(_BF),
                      preferred_element_type=jnp.float32) + b1_ref[j][None, :]
        hid = jnp.maximum(hid, 0.0)
        eo = jnp.dot(hid.astype(_BF), w2_ref[j].astype(_BF),
                     preferred_element_type=jnp.float32) + b2_ref[j][None, :]
        acc = acc + g[:, j:j + 1] * eo
    y_ref[...] = jnp.dot(acc.astype(_BF), clswt_ref[...],
                         preferred_element_type=jnp.float32) + clsb_ref[...]


def kernel(x, conv1_w, conv1_b, conv2_w, conv2_b, proj_w, proj_b,
           router_w, router_b, w1, b1, w2, b2, cls_w, cls_b):
    f32 = jnp.float32
    x = x.astype(f32)
    xpad = jnp.pad(x, ((0, 0), (2, 14))).astype(_BF)
    xa = xpad.reshape(B, 65, 16)
    c1 = jnp.einsum('pjk,ck->pjc', jnp.asarray(_E1),
                    conv1_w[:, 0, :]).reshape(24, 8 * CONV_DIM).astype(_BF)
    b1big = jnp.tile(conv1_b, 8)[None, :]
    w2f = jnp.concatenate([
        jnp.einsum('qrk,dck->qcrd', jnp.asarray(_EA),
                   conv2_w).reshape(8 * CONV_DIM, 4 * CONV_DIM),
        jnp.einsum('qrk,dck->qcrd', jnp.asarray(_EB),
                   conv2_w).reshape(2 * CONV_DIM, 4 * CONV_DIM),
        jnp.einsum('qrk,dck->qcrd', jnp.asarray(_EC),
                   conv2_w).reshape(CONV_DIM, 4 * CONV_DIM),
    ], axis=0).astype(_BF)
    b2big = jnp.tile(conv2_b, 4)[None, :]
    projwt = proj_w.T.astype(_BF)
    projbr = proj_b[None, :]
    routwt = jnp.pad(router_w.T, ((0, 0), (0, 2))).astype(_BF)
    routb = jnp.pad(router_b, (0, 2), constant_values=-1e30)[None, :]

    grid1 = B // _R1
    h, g = pl.pallas_call(
        _frontend_kernel,
        grid=(grid1,),
        in_specs=[
            pl.BlockSpec((_R1, 65, 16), lambda i: (i, 0, 0)),
            pl.BlockSpec((24, 8 * CONV_DIM), lambda i: (0, 0)),
            pl.BlockSpec((1, 8 * CONV_DIM), lambda i: (0, 0)),
            pl.BlockSpec((11 * CONV_DIM, 4 * CONV_DIM), lambda i: (0, 0)),
            pl.BlockSpec((1, 4 * CONV_DIM), lambda i: (0, 0)),
            pl.BlockSpec((CONV_DIM, MOE_DIM), lambda i: (0, 0)),
            pl.BlockSpec((1, MOE_DIM), lambda i: (0, 0)),
            pl.BlockSpec((MOE_DIM, 8), lambda i: (0, 0)),
            pl.BlockSpec((1, 8), lambda i: (0, 0)),
        ],
        out_specs=[
            pl.BlockSpec((_R1, MOE_DIM), lambda i: (i, 0)),
            pl.BlockSpec((_R1, 8), lambda i: (i, 0)),
        ],
        out_shape=[
            jax.ShapeDtypeStruct((B, MOE_DIM), f32),
            jax.ShapeDtypeStruct((B, 8), f32),
        ],
    )(xa, c1, b1big, w2f, b2big, projwt, projbr, routwt, routb)

    grid2 = B // _R2
    y = pl.pallas_call(
        _moe_kernel,
        grid=(grid2,),
        in_specs=[
            pl.BlockSpec((_R2, MOE_DIM), lambda i: (i, 0)),
            pl.BlockSpec((_R2, 8), lambda i: (i, 0)),
            pl.BlockSpec((E, MOE_DIM, FF_DIM), lambda i: (0, 0, 0)),
            pl.BlockSpec((E, FF_DIM), lambda i: (0, 0)),
            pl.BlockSpec((E, FF_DIM, MOE_DIM), lambda i: (0, 0, 0)),
            pl.BlockSpec((E, MOE_DIM), lambda i: (0, 0)),
            pl.BlockSpec((MOE_DIM, NUM_CLASSES), lambda i: (0, 0)),
            pl.BlockSpec((1, NUM_CLASSES), lambda i: (0, 0)),
        ],
        out_specs=pl.BlockSpec((_R2, NUM_CLASSES), lambda i: (i, 0)),
        out_shape=jax.ShapeDtypeStruct((B, NUM_CLASSES), f32),
    )(h, g, w1, b1, w2, b2,
      cls_w.T.astype(_BF), cls_b[None, :])
    return y
